# baseline (speedup 1.0000x reference)
import functools

import jax
import jax.numpy as jnp
from jax.experimental import pallas as pl
from jax.experimental.pallas import tpu as pltpu

_BS, _A, _H, _W, _NC = 64, 3, 52, 52, 80
_ATTRS = 5 + _NC


def _softplus(z):
    return jnp.maximum(z, 0.0) + jnp.log(1.0 + jnp.exp(-jnp.abs(z)))


def _loss_kernel(pred_ref, tcls_ref, mask_ref, noobj_ref, tx_ref, ty_ref,
                 tw_ref, th_ref, bsx_ref, bsy_ref, out_ref, acc_ref):
    i = pl.program_id(0)

    @pl.when(i == 0)
    def _init():
        acc_ref[...] = jnp.zeros_like(acc_ref)

    p = pred_ref[0]
    acc = acc_ref[...]
    for a in range(_A):
        c0 = _ATTRS * a
        m = mask_ref[a, 0]
        nm = noobj_ref[a, 0]
        t_x = tx_ref[a, 0]
        t_y = ty_ref[a, 0]
        t_w = tw_ref[a, 0]
        t_h = th_ref[a, 0]
        sx = bsx_ref[a, 0]
        sy = bsy_ref[a, 0]

        zx = p[:, :, c0 + 0]
        zy = p[:, :, c0 + 1]
        zw = p[:, :, c0 + 2]
        zh = p[:, :, c0 + 3]
        zc = p[:, :, c0 + 4]
        zcls = p[:, :, c0 + 5:c0 + _ATTRS]
        tcls_a = tcls_ref[a, 0]

        dw = zw - t_w
        dh = zh - t_h
        box = (_softplus(zx) - t_x * zx) + (_softplus(zy) - t_y * zy) \
            + dw * dw + dh * dh
        plane = box * ((2.0 - sx * sy) * m)
        plane += (_softplus(zc) - m * zc) * (m + nm)

        cls_term = jnp.sum(_softplus(zcls) - tcls_a * zcls, axis=2)
        plane += cls_term * m
        acc += plane
    acc_ref[...] = acc

    @pl.when(i == pl.num_programs(0) - 1)
    def _finish():
        out_ref[0, 0] = jnp.sum(acc_ref[...]) * (1.0 / _BS)


@functools.partial(jax.jit, static_argnames=("interpret",))
def kernel(pred, mask, noobj_mask, tx, ty, tw, th, tcls,
           box_loss_scale_x, box_loss_scale_y, interpret=False):
    pred_t = jnp.transpose(pred, (2, 3, 0, 1))
    tcls_t = jnp.transpose(tcls, (1, 2, 3, 0, 4))
    tp = lambda v: jnp.transpose(v, (1, 2, 3, 0))

    plane = pl.BlockSpec((_A, 1, _W, _BS), lambda i: (0, i, 0, 0))
    out = pl.pallas_call(
        _loss_kernel,
        grid=(_H,),
        in_specs=[
            pl.BlockSpec((1, _W, _BS, _A * _ATTRS), lambda i: (i, 0, 0, 0)),
            pl.BlockSpec((_A, 1, _W, _BS, _NC), lambda i: (0, i, 0, 0, 0)),
            plane, plane, plane, plane, plane, plane, plane, plane,
        ],
        out_specs=pl.BlockSpec(
            (1, 1), lambda i: (0, 0), memory_space=pltpu.SMEM),
        out_shape=jax.ShapeDtypeStruct((1, 1), jnp.float32),
        scratch_shapes=[pltpu.VMEM((_W, _BS), jnp.float32)],
        interpret=interpret,
    )(pred_t, tcls_t, tp(mask), tp(noobj_mask), tp(tx), tp(ty),
      tp(tw), tp(th), tp(box_loss_scale_x), tp(box_loss_scale_y))
    return out[0, 0]


# speedup vs baseline: 2.0002x; 2.0002x over previous
import functools

import jax
import jax.numpy as jnp
from jax.experimental import pallas as pl
from jax.experimental.pallas import tpu as pltpu

_BS, _A, _H, _W, _NC = 64, 3, 52, 52, 80
_ATTRS = 5 + _NC


def _softplus(z):
    return jnp.maximum(z, 0.0) + jnp.log(1.0 + jnp.exp(-jnp.abs(z)))


def _loss_kernel(pred_ref, mask_ref, noobj_ref, tx_ref, ty_ref, tw_ref,
                 th_ref, tcls_ref, bsx_ref, bsy_ref, out_ref, acc_ref):
    b = pl.program_id(0)

    @pl.when(b == 0)
    def _init():
        acc_ref[...] = jnp.zeros_like(acc_ref)

    acc = acc_ref[...]
    for a in range(_A):
        m = mask_ref[0, a]
        nm = noobj_ref[0, a]
        t_x = tx_ref[0, a]
        t_y = ty_ref[0, a]
        t_w = tw_ref[0, a]
        t_h = th_ref[0, a]
        sx = bsx_ref[0, a]
        sy = bsy_ref[0, a]

        c0 = _ATTRS * a
        zx = pred_ref[0, c0 + 0]
        zy = pred_ref[0, c0 + 1]
        zw = pred_ref[0, c0 + 2]
        zh = pred_ref[0, c0 + 3]
        zc = pred_ref[0, c0 + 4]
        zcls = pred_ref[0, c0 + 5:c0 + _ATTRS]

        dw = zw - t_w
        dh = zh - t_h
        box = (_softplus(zx) - t_x * zx) + (_softplus(zy) - t_y * zy) \
            + dw * dw + dh * dh
        plane = box * ((2.0 - sx * sy) * m)
        plane += (_softplus(zc) - m * zc) * (m + nm)

        tcls_t = jnp.transpose(tcls_ref[0, a], (2, 0, 1))
        cls_term = jnp.sum(_softplus(zcls) - tcls_t * zcls, axis=0)
        plane += cls_term * m
        acc += plane
    acc_ref[...] = acc

    @pl.when(b == _BS - 1)
    def _finish():
        out_ref[0, 0] = jnp.sum(acc_ref[...]) * (1.0 / _BS)


@functools.partial(jax.jit, static_argnames=("interpret",))
def kernel(pred, mask, noobj_mask, tx, ty, tw, th, tcls,
           box_loss_scale_x, box_loss_scale_y, interpret=False):
    plane = pl.BlockSpec((1, _A, _H, _W), lambda b: (b, 0, 0, 0))
    out = pl.pallas_call(
        _loss_kernel,
        grid=(_BS,),
        in_specs=[
            pl.BlockSpec((1, _A * _ATTRS, _H, _W), lambda b: (b, 0, 0, 0)),
            plane, plane, plane, plane, plane, plane,
            pl.BlockSpec((1, _A, _H, _W, _NC), lambda b: (b, 0, 0, 0, 0)),
            plane, plane,
        ],
        out_specs=pl.BlockSpec(
            (1, 1), lambda b: (0, 0), memory_space=pltpu.SMEM),
        out_shape=jax.ShapeDtypeStruct((1, 1), jnp.float32),
        scratch_shapes=[pltpu.VMEM((_H, _W), jnp.float32)],
        interpret=interpret,
    )(pred, mask, noobj_mask, tx, ty, tw, th, tcls,
      box_loss_scale_x, box_loss_scale_y)
    return out[0, 0]
